# baseline (speedup 1.0000x reference)
import functools

import jax
import jax.numpy as jnp
import numpy as np
from jax import lax
from jax.experimental import pallas as pl
from jax.experimental.pallas import tpu as pltpu
from jax.experimental.pallas import tpu_sc as plsc

NUM_SEG = 128
B = 16384
NS = 16
L = 16
PER_W = B // NS

_mesh = plsc.VectorSubcoreMesh(
    core_axis_name="c", subcore_axis_name="s", num_cores=1, num_subcores=NS)
_params = pltpu.CompilerParams(needs_layout_passes=False)


@functools.partial(
    pl.kernel,
    out_type=jax.ShapeDtypeStruct((L,), jnp.float32),
    mesh=_mesh,
    scratch_types=[
        pltpu.VMEM((PER_W,), jnp.float32),
        pltpu.VMEM((PER_W,), jnp.int32),
        pltpu.VMEM((PER_W,), jnp.int32),
        pltpu.VMEM((L * NUM_SEG,), jnp.int32),
        pltpu.VMEM((1, 2 * NUM_SEG), jnp.float32),
        pltpu.VMEM((1,), jnp.int32),
        pltpu.VMEM_SHARED((1, 2 * NUM_SEG), jnp.float32),
        pltpu.VMEM((1, 2 * NUM_SEG), jnp.float32),
        pltpu.VMEM((L,), jnp.float32),
        pltpu.SemaphoreType.DMA,
        pltpu.SemaphoreType.DMA,
        pltpu.SemaphoreType.DMA,
    ],
    compiler_params=_params,
)
def _sc_loss(probs_hbm, labels_hbm, groups_hbm, zeros_hbm, out_hbm,
             probs_v, labels_v, groups_v, bins_b, part_v, idx0_v,
             shared_sp, tot_v, res_v, sem1, sem2, sem3):
    wid = lax.axis_index("s")
    base = wid * PER_W

    zeros = jnp.zeros((L,), jnp.float32)
    izeros = jnp.zeros((L,), jnp.int32)
    lane = lax.iota(jnp.int32, L)

    c1 = pltpu.async_copy(probs_hbm.at[pl.ds(base, PER_W)], probs_v, sem1)
    c2 = pltpu.async_copy(labels_hbm.at[pl.ds(base, PER_W)], labels_v, sem2)
    c3 = pltpu.async_copy(groups_hbm.at[pl.ds(base, PER_W)], groups_v, sem3)
    pltpu.sync_copy(zeros_hbm.at[pl.ds(0, 1)], idx0_v)

    @pl.when(wid == 0)
    def _():
        def _zpart(k, carry):
            part_v[0, pl.ds(k * L, L)] = zeros
            return carry
        lax.fori_loop(0, 2 * NUM_SEG // L, _zpart, 0, unroll=1)
        pltpu.sync_copy(part_v, shared_sp)

    def _zero(i, carry):
        bins_b[pl.ds(i * L, L)] = izeros
        return carry
    lax.fori_loop(0, L * NUM_SEG // L, _zero, 0, unroll=1)

    plsc.subcore_barrier()
    c1.wait()
    c2.wait()
    c3.wait()

    half = jnp.full((L,), 0.5, jnp.float32)
    scale = jnp.full((L,), 2048.0, jnp.float32)
    cnt_one = jnp.full((L,), 1 << 20, jnp.int32)

    def _accum(i, carry):
        p = probs_v[pl.ds(i * L, L)]
        lbl = labels_v[pl.ds(i * L, L)]
        grp = groups_v[pl.ds(i * L, L)]
        q = (p * scale + half).astype(jnp.int32)
        seg = grp + 64 * lbl
        flat = lane * NUM_SEG + seg
        plsc.addupdate_scatter(bins_b, [flat], q + cnt_one)
        return carry
    lax.fori_loop(0, PER_W // L, _accum, 0, unroll=1)

    qmask = jnp.full((L,), (1 << 20) - 1, jnp.int32)
    shift20 = jnp.full((L,), 20, jnp.int32)
    inv_scale = jnp.full((L,), 1.0 / 2048.0, jnp.float32)

    def _reduce(j, carry):
        def _lane(l, acc):
            return acc + bins_b[pl.ds(l * NUM_SEG + j * L, L)]
        acc = lax.fori_loop(0, L, _lane, izeros, unroll=2)
        cnt_i = lax.shift_right_logical(acc, shift20)
        sq_i = acc & qmask
        part_v[0, pl.ds(j * L, L)] = sq_i.astype(jnp.float32) * inv_scale
        part_v[0, pl.ds(NUM_SEG + j * L, L)] = cnt_i.astype(jnp.float32)
        return carry
    lax.fori_loop(0, NUM_SEG // L, _reduce, 0, unroll=1)

    pltpu.sync_copy(part_v, shared_sp.at[idx0_v], add=True)
    plsc.subcore_barrier()

    @pl.when(wid == 0)
    def _():
        pltpu.sync_copy(shared_sp, tot_v)

        ones_v = jnp.ones((L,), jnp.float32)

        def _half_stats(h, carry):
            seg_base = h * 64

            def _pass1(j, c1):
                acc_nv, acc_m = c1
                s = tot_v[0, pl.ds(seg_base + j * L, L)]
                c = tot_v[0, pl.ds(NUM_SEG + seg_base + j * L, L)]
                valid = c >= 1.0
                m = s / jnp.maximum(c, ones_v)
                return (acc_nv + jnp.where(valid, 1.0, 0.0),
                        acc_m + jnp.where(valid, m, 0.0))
            acc_nv, acc_m = lax.fori_loop(0, 64 // L, _pass1, (zeros, zeros))
            nv = jnp.sum(acc_nv)
            nv_v = jnp.full((L,), nv)
            mom_v = jnp.full((L,), jnp.sum(acc_m)) / jnp.maximum(nv_v, ones_v)

            def _pass2(j, acc_var):
                s = tot_v[0, pl.ds(seg_base + j * L, L)]
                c = tot_v[0, pl.ds(NUM_SEG + seg_base + j * L, L)]
                valid = c >= 1.0
                m = s / jnp.maximum(c, ones_v)
                d = m - mom_v
                return acc_var + jnp.where(valid, d * d, 0.0)
            acc_var = lax.fori_loop(0, 64 // L, _pass2, zeros)
            var_v = (jnp.full((L,), jnp.sum(acc_var))
                     / jnp.maximum(nv_v - ones_v, ones_v))
            _, _, cur_var, cur_nv = carry
            return (cur_var, cur_nv, var_v, nv_v)

        neg_var_v, nv_neg_v, pos_var_v, nv_pos_v = lax.fori_loop(
            0, 2, _half_stats, (zeros, zeros, zeros, zeros))

        has_pos = nv_pos_v >= 2.0
        has_neg = nv_neg_v >= 2.0
        total_v = jnp.where(
            has_pos & has_neg,
            (pos_var_v + neg_var_v) * 0.5,
            jnp.where(has_pos, pos_var_v,
                      jnp.where(has_neg, neg_var_v, zeros)),
        )
        res_v[...] = total_v
        pltpu.sync_copy(res_v, out_hbm)


_ZEROS_I32 = np.zeros((8,), np.int32)


def kernel(probs, labels, groups):
    probs = probs.reshape(-1)
    out = _sc_loss(probs, labels, groups, _ZEROS_I32)
    return out[0]


# speedup vs baseline: 1.0178x; 1.0178x over previous
import functools

import jax
import jax.numpy as jnp
import numpy as np
from jax import lax
from jax.experimental import pallas as pl
from jax.experimental.pallas import tpu as pltpu
from jax.experimental.pallas import tpu_sc as plsc

NUM_SEG = 128
B = 16384
NS = 16
L = 16
PER_W = B // NS

_mesh = plsc.VectorSubcoreMesh(
    core_axis_name="c", subcore_axis_name="s", num_cores=1, num_subcores=NS)
_params = pltpu.CompilerParams(needs_layout_passes=False)


@functools.partial(
    pl.kernel,
    out_type=jax.ShapeDtypeStruct((L,), jnp.float32),
    mesh=_mesh,
    scratch_types=[
        pltpu.VMEM((PER_W,), jnp.float32),
        pltpu.VMEM((PER_W,), jnp.int32),
        pltpu.VMEM((PER_W,), jnp.int32),
        pltpu.VMEM((L * NUM_SEG,), jnp.int32),
        pltpu.VMEM((1, 2 * NUM_SEG), jnp.float32),
        pltpu.VMEM((1,), jnp.int32),
        pltpu.VMEM_SHARED((1, 2 * NUM_SEG), jnp.float32),
        pltpu.VMEM((1, 2 * NUM_SEG), jnp.float32),
        pltpu.VMEM((L,), jnp.float32),
        pltpu.SemaphoreType.DMA,
        pltpu.SemaphoreType.DMA,
        pltpu.SemaphoreType.DMA,
    ],
    compiler_params=_params,
)
def _sc_loss(probs_hbm, labels_hbm, groups_hbm, zeros_hbm, out_hbm,
             probs_v, labels_v, groups_v, bins_b, part_v, idx0_v,
             shared_sp, tot_v, res_v, sem1, sem2, sem3):
    wid = lax.axis_index("s")
    base = wid * PER_W

    zeros = jnp.zeros((L,), jnp.float32)
    izeros = jnp.zeros((L,), jnp.int32)
    lane = lax.iota(jnp.int32, L)

    c1 = pltpu.async_copy(probs_hbm.at[pl.ds(base, PER_W)], probs_v, sem1)
    c2 = pltpu.async_copy(labels_hbm.at[pl.ds(base, PER_W)], labels_v, sem2)
    c3 = pltpu.async_copy(groups_hbm.at[pl.ds(base, PER_W)], groups_v, sem3)
    pltpu.sync_copy(zeros_hbm.at[pl.ds(0, 1)], idx0_v)

    @pl.when(wid == 0)
    def _():
        def _zpart(k, carry):
            part_v[0, pl.ds(k * L, L)] = zeros
            return carry
        lax.fori_loop(0, 2 * NUM_SEG // L, _zpart, 0, unroll=8)
        pltpu.sync_copy(part_v, shared_sp)

    def _zero(i, carry):
        bins_b[pl.ds(i * L, L)] = izeros
        return carry
    lax.fori_loop(0, L * NUM_SEG // L, _zero, 0, unroll=4)

    plsc.subcore_barrier()
    c1.wait()
    c2.wait()
    c3.wait()

    half = jnp.full((L,), 0.5, jnp.float32)
    scale = jnp.full((L,), 2048.0, jnp.float32)
    cnt_one = jnp.full((L,), 1 << 20, jnp.int32)

    def _accum(i, carry):
        p = probs_v[pl.ds(i * L, L)]
        lbl = labels_v[pl.ds(i * L, L)]
        grp = groups_v[pl.ds(i * L, L)]
        q = (p * scale + half).astype(jnp.int32)
        seg = grp + 64 * lbl
        flat = lane * NUM_SEG + seg
        plsc.addupdate_scatter(bins_b, [flat], q + cnt_one)
        return carry
    lax.fori_loop(0, PER_W // L, _accum, 0, unroll=2)

    qmask = jnp.full((L,), (1 << 20) - 1, jnp.int32)
    shift20 = jnp.full((L,), 20, jnp.int32)
    inv_scale = jnp.full((L,), 1.0 / 2048.0, jnp.float32)

    def _reduce(j, carry):
        def _lane(l, acc):
            return acc + bins_b[pl.ds(l * NUM_SEG + j * L, L)]
        acc = lax.fori_loop(0, L, _lane, izeros, unroll=8)
        cnt_i = lax.shift_right_logical(acc, shift20)
        sq_i = acc & qmask
        part_v[0, pl.ds(j * L, L)] = sq_i.astype(jnp.float32) * inv_scale
        part_v[0, pl.ds(NUM_SEG + j * L, L)] = cnt_i.astype(jnp.float32)
        return carry
    lax.fori_loop(0, NUM_SEG // L, _reduce, 0, unroll=1)

    pltpu.sync_copy(part_v, shared_sp.at[idx0_v], add=True)
    plsc.subcore_barrier()

    @pl.when(wid == 0)
    def _():
        pltpu.sync_copy(shared_sp, tot_v)

        ones_v = jnp.ones((L,), jnp.float32)

        def _half_stats(h, carry):
            seg_base = h * 64

            def _pass1(j, c1):
                acc_nv, acc_m = c1
                s = tot_v[0, pl.ds(seg_base + j * L, L)]
                c = tot_v[0, pl.ds(NUM_SEG + seg_base + j * L, L)]
                valid = c >= 1.0
                m = s / jnp.maximum(c, ones_v)
                return (acc_nv + jnp.where(valid, 1.0, 0.0),
                        acc_m + jnp.where(valid, m, 0.0))
            acc_nv, acc_m = lax.fori_loop(0, 64 // L, _pass1, (zeros, zeros))
            nv = jnp.sum(acc_nv)
            nv_v = jnp.full((L,), nv)
            mom_v = jnp.full((L,), jnp.sum(acc_m)) / jnp.maximum(nv_v, ones_v)

            def _pass2(j, acc_var):
                s = tot_v[0, pl.ds(seg_base + j * L, L)]
                c = tot_v[0, pl.ds(NUM_SEG + seg_base + j * L, L)]
                valid = c >= 1.0
                m = s / jnp.maximum(c, ones_v)
                d = m - mom_v
                return acc_var + jnp.where(valid, d * d, 0.0)
            acc_var = lax.fori_loop(0, 64 // L, _pass2, zeros)
            var_v = (jnp.full((L,), jnp.sum(acc_var))
                     / jnp.maximum(nv_v - ones_v, ones_v))
            _, _, cur_var, cur_nv = carry
            return (cur_var, cur_nv, var_v, nv_v)

        neg_var_v, nv_neg_v, pos_var_v, nv_pos_v = lax.fori_loop(
            0, 2, _half_stats, (zeros, zeros, zeros, zeros))

        has_pos = nv_pos_v >= 2.0
        has_neg = nv_neg_v >= 2.0
        total_v = jnp.where(
            has_pos & has_neg,
            (pos_var_v + neg_var_v) * 0.5,
            jnp.where(has_pos, pos_var_v,
                      jnp.where(has_neg, neg_var_v, zeros)),
        )
        res_v[...] = total_v
        pltpu.sync_copy(res_v, out_hbm)


_ZEROS_I32 = np.zeros((8,), np.int32)


def kernel(probs, labels, groups):
    probs = probs.reshape(-1)
    out = _sc_loss(probs, labels, groups, _ZEROS_I32)
    return out[0]
